# baseline (speedup 1.0000x reference)
import functools

import jax
import jax.numpy as jnp
from jax import lax
from jax.experimental import pallas as pl
from jax.experimental.pallas import tpu as pltpu
from jax.experimental.pallas import tpu_sc as plsc

TOTAL_ROWS = 16384 * 26
EMBED_DIM = 64
OUT_DIM = 10
NUM_WORKERS = 32
ROWS_PER_W = TOTAL_ROWS // NUM_WORKERS
CHUNK = 128
CHUNKS_PER_W = ROWS_PER_W // CHUNK
FIRE = 8
GROUPS = CHUNKS_PER_W // FIRE
GROUP_ROWS = FIRE * CHUNK
MM_BLK = 4096
MM_GRID = TOTAL_ROWS // MM_BLK


def _gather_body(emb_hbm, idx_hbm, out_hbm, idx_v, buf_v, sem):
    wid = lax.axis_index("s") * 2 + lax.axis_index("c")
    pltpu.sync_copy(idx_hbm.at[wid], idx_v)
    base = wid * ROWS_PER_W

    def group(g, carry):
        copies = []
        for i in range(FIRE):
            c = pltpu.async_copy(
                emb_hbm.at[idx_v.at[g * FIRE + i]],
                buf_v.at[pl.ds(i * CHUNK, CHUNK)],
                sem,
            )
            copies.append(c)
        for c in copies:
            c.wait()
        pltpu.sync_copy(buf_v, out_hbm.at[pl.ds(base + g * GROUP_ROWS, GROUP_ROWS)])
        return carry

    lax.fori_loop(0, GROUPS, group, 0)


_sc_gather = pl.kernel(
    _gather_body,
    out_type=jax.ShapeDtypeStruct((TOTAL_ROWS, EMBED_DIM), jnp.float32),
    mesh=plsc.VectorSubcoreMesh(core_axis_name="c", subcore_axis_name="s"),
    scratch_types=[
        pltpu.VMEM((CHUNKS_PER_W, CHUNK), jnp.int32),
        pltpu.VMEM((GROUP_ROWS, EMBED_DIM), jnp.float32),
        pltpu.SemaphoreType.DMA,
    ],
    compiler_params=pltpu.CompilerParams(use_tc_tiling_on_sc=False),
)


def _mm_body(x_ref, w_ref, b_ref, o_ref):
    o_ref[...] = (
        lax.dot_general(
            x_ref[...], w_ref[...],
            (((1,), (1,)), ((), ())),
            preferred_element_type=jnp.float32,
        )
        + b_ref[...]
    )


_tc_matmul = pl.pallas_call(
    _mm_body,
    grid=(MM_GRID,),
    in_specs=[
        pl.BlockSpec((MM_BLK, EMBED_DIM), lambda i: (i, 0)),
        pl.BlockSpec((OUT_DIM, EMBED_DIM), lambda i: (0, 0)),
        pl.BlockSpec((1, OUT_DIM), lambda i: (0, 0)),
    ],
    out_specs=pl.BlockSpec((MM_BLK, OUT_DIM), lambda i: (i, 0)),
    out_shape=jax.ShapeDtypeStruct((TOTAL_ROWS, OUT_DIM), jnp.float32),
)


def kernel(x_id, emb, W, b):
    batch, fields = x_id.shape
    idx = x_id.astype(jnp.int32).reshape(NUM_WORKERS, CHUNKS_PER_W, CHUNK)
    rows = _sc_gather(emb, idx)
    out = _tc_matmul(rows, W, b.reshape(1, OUT_DIM))
    return out.reshape(batch, fields, OUT_DIM)


# speedup vs baseline: 1.1323x; 1.1323x over previous
import functools

import jax
import jax.numpy as jnp
from jax import lax
from jax.experimental import pallas as pl
from jax.experimental.pallas import tpu as pltpu
from jax.experimental.pallas import tpu_sc as plsc

TOTAL_ROWS = 16384 * 26
EMBED_DIM = 64
PAD_DIM = 128
OUT_DIM = 10
NUM_WORKERS = 32
ROWS_PER_W = TOTAL_ROWS // NUM_WORKERS
CHUNK = 128
CHUNKS_PER_W = ROWS_PER_W // CHUNK
FIRE = 8
GROUPS = CHUNKS_PER_W // FIRE
GROUP_ROWS = FIRE * CHUNK
MM_BLK = 4096
MM_GRID = TOTAL_ROWS // MM_BLK


def _gather_body(emb_hbm, idx_hbm, out_hbm, idx_v, buf_v, sem):
    wid = lax.axis_index("s") * 2 + lax.axis_index("c")
    pltpu.sync_copy(idx_hbm.at[wid], idx_v)
    base = wid * ROWS_PER_W

    def group(g, carry):
        copies = []
        for i in range(FIRE):
            c = pltpu.async_copy(
                emb_hbm.at[idx_v.at[g * FIRE + i]],
                buf_v.at[pl.ds(i * CHUNK, CHUNK)],
                sem,
            )
            copies.append(c)
        for c in copies:
            c.wait()
        pltpu.sync_copy(
            buf_v,
            out_hbm.at[pl.ds(base + g * GROUP_ROWS, GROUP_ROWS), pl.ds(0, EMBED_DIM)],
        )
        return carry

    lax.fori_loop(0, GROUPS, group, 0)


_sc_gather = pl.kernel(
    _gather_body,
    out_type=jax.ShapeDtypeStruct((TOTAL_ROWS, PAD_DIM), jnp.float32),
    mesh=plsc.VectorSubcoreMesh(core_axis_name="c", subcore_axis_name="s"),
    scratch_types=[
        pltpu.VMEM((CHUNKS_PER_W, CHUNK), jnp.int32),
        pltpu.VMEM((GROUP_ROWS, EMBED_DIM), jnp.float32),
        pltpu.SemaphoreType.DMA,
    ],
    compiler_params=pltpu.CompilerParams(use_tc_tiling_on_sc=False),
)


def _mm_body(x_ref, w_ref, b_ref, o_ref):
    o_ref[...] = (
        lax.dot_general(
            x_ref[:, :EMBED_DIM], w_ref[...],
            (((1,), (1,)), ((), ())),
            preferred_element_type=jnp.float32,
        )
        + b_ref[...]
    )


_tc_matmul = pl.pallas_call(
    _mm_body,
    grid=(MM_GRID,),
    in_specs=[
        pl.BlockSpec((MM_BLK, PAD_DIM), lambda i: (i, 0)),
        pl.BlockSpec((OUT_DIM, EMBED_DIM), lambda i: (0, 0)),
        pl.BlockSpec((1, OUT_DIM), lambda i: (0, 0)),
    ],
    out_specs=pl.BlockSpec((MM_BLK, OUT_DIM), lambda i: (i, 0)),
    out_shape=jax.ShapeDtypeStruct((TOTAL_ROWS, OUT_DIM), jnp.float32),
)


def kernel(x_id, emb, W, b):
    batch, fields = x_id.shape
    idx = x_id.astype(jnp.int32).reshape(NUM_WORKERS, CHUNKS_PER_W, CHUNK)
    rows = _sc_gather(emb, idx)
    out = _tc_matmul(rows, W, b.reshape(1, OUT_DIM))
    return out.reshape(batch, fields, OUT_DIM)
